# baseline (speedup 1.0000x reference)
import functools

import jax
import jax.numpy as jnp
from jax import lax
from jax.experimental import pallas as pl
from jax.experimental.pallas import tpu as pltpu
from jax.experimental.pallas import tpu_sc as plsc

_NUM_POIS = 1000000
_EMBED_DIM = 64
_BATCH = 4096
_HIST = 50

_NC = 2
_NS = 16
_NW = _NC * _NS
_BT = _BATCH // _NW

_mesh = plsc.VectorSubcoreMesh(core_axis_name="c", subcore_axis_name="s")
_params = pltpu.CompilerParams(use_tc_tiling_on_sc=True,
                               needs_layout_passes=False)


@functools.partial(
    pl.kernel,
    mesh=_mesh,
    out_type=jax.ShapeDtypeStruct((_HIST, _EMBED_DIM, _BATCH), jnp.float32),
    scratch_types=[
        pltpu.VMEM((4, 1, _BT), jnp.int32),
        pltpu.VMEM((4, 1, _BT), jnp.int32),
        pltpu.VMEM((4, 1, _BT), jnp.int32),
        pltpu.VMEM((4, _BT, 128), jnp.float32),
        pltpu.VMEM((2, _EMBED_DIM, _BT), jnp.float32),
        pltpu.SemaphoreType.DMA((4,)),
        pltpu.SemaphoreType.DMA((4,)),
        pltpu.SemaphoreType.DMA((2,)),
    ],
    compiler_params=_params,
)
def _sc_gather(idx_t, lin, out3, raw_v, gidx_v, par_v, rows_v, stage_v,
               isem, gsem, wsem):
    wid = lax.axis_index("s") * _NC + lax.axis_index("c")
    b0 = pl.multiple_of(wid * _BT, 128)
    lanes = lax.iota(jnp.int32, 16)
    lanes_c0 = [lanes + (16 * g) for g in range(_BT // 16)]

    def idx_fetch(h, buf):
        pltpu.async_copy(idx_t.at[h, pl.ds(b0, _BT)], raw_v.at[buf, 0],
                         isem.at[buf])

    def idx_wait(buf):
        pltpu.make_async_copy(idx_t.at[0, pl.ds(0, _BT)], raw_v.at[buf, 0],
                              isem.at[buf]).wait()

    def idx_split(buf):
        for g in range(_BT // 16):
            v = raw_v[buf, 0, pl.ds(16 * g, 16)]
            gidx_v[buf, 0, pl.ds(16 * g, 16)] = v >> 1
            par_v[buf, 0, pl.ds(16 * g, 16)] = v & 1

    def gather(buf):
        pltpu.async_copy(lin.at[gidx_v.at[buf, 0]], rows_v.at[buf],
                         gsem.at[buf])

    def gather_wait(buf):
        pltpu.make_async_copy(lin.at[gidx_v.at[buf, 0]], rows_v.at[buf],
                              gsem.at[buf]).wait()

    def transpose(buf, sbuf):
        for g in range(_BT // 16):
            vpre = par_v[buf, 0, pl.ds(16 * g, 16)] * _EMBED_DIM
            for d0 in range(0, _EMBED_DIM, 8):
                vs = [plsc.load_gather(rows_v.at[buf],
                                       [lanes_c0[g], vpre + (d0 + j)])
                      for j in range(8)]
                for j in range(8):
                    stage_v[sbuf, d0 + j, pl.ds(16 * g, 16)] = vs[j]

    def writeback(h, sbuf):
        pltpu.async_copy(stage_v.at[sbuf], out3.at[h, :, pl.ds(b0, _BT)],
                         wsem.at[sbuf])

    def writeback_wait(sbuf):
        pltpu.make_async_copy(stage_v.at[sbuf], out3.at[0, :, pl.ds(0, _BT)],
                              wsem.at[sbuf]).wait()

    for b in range(4):
        idx_fetch(b, b)
    for b in range(3):
        idx_wait(b)
        idx_split(b)
        gather(b)

    def step(t, carry):
        for q in range(4):
            h = 4 * t + q
            gather_wait(q)

            qn = (q + 3) % 4

            def _advance():
                idx_wait(qn)
                idx_split(qn)
                gather(qn)
            pl.when(h + 3 < _HIST)(_advance)
            pl.when(h + 4 < _HIST)(lambda: idx_fetch(h + 4, q))
            pl.when(h >= 2)(lambda: writeback_wait(q % 2))
            transpose(q, q % 2)
            writeback(h, q % 2)
        return carry

    lax.fori_loop(0, (_HIST - 2) // 4, step, 0)
    for h, q in ((48, 0), (49, 1)):
        gather_wait(q)
        writeback_wait(q % 2)
        transpose(q, q % 2)
        writeback(h, q % 2)
    writeback_wait(0)
    writeback_wait(1)


def kernel(poi_idx, geo_embedding_weight):
    lin = jnp.reshape(geo_embedding_weight, (_NUM_POIS // 2, 2 * _EMBED_DIM))
    out3 = _sc_gather(poi_idx.T.astype(jnp.int32), lin)
    return jnp.transpose(out3, (2, 0, 1))
